# baseline (speedup 1.0000x reference)
import functools

import jax
import jax.numpy as jnp
from jax import lax
from jax.experimental import pallas as pl
from jax.experimental.pallas import tpu as pltpu
from jax.experimental.pallas import tpu_sc as plsc

N = 10000
E = 160000
D = 256
DE = 16
H = 512
G = 64
T = 10

DH = D // 2
N_PAD = 10112
ROWS_PER_TILE = N_PAD // 16
CHUNK = 64
E_PAD = 163840
N_CHUNKS = E_PAD // CHUNK
N_TILES = 16
CPT = N_CHUNKS // N_TILES


def _ilv(u):
    lb = lax.bitcast_convert_type(u[:, :64], jnp.int32)
    hb = lax.bitcast_convert_type(u[:, 64:], jnp.int32)
    lr = lb + 0x7FFF + ((lb >> 16) & 1)
    hr = hb + 0x7FFF + ((hb >> 16) & 1)
    return (hr & jnp.int32(-65536)) | ((lr >> 16) & 0xFFFF)


def _edge_proj_body(ea_ref, we0_ref, we1_ref, o00, o01, o10, o11):
    ea = ea_ref[...]
    p = jnp.dot(ea, we0_ref[...], preferred_element_type=jnp.float32)
    o00[...] = _ilv(p[:, :DH])
    o01[...] = _ilv(p[:, DH:])
    p = jnp.dot(ea, we1_ref[...], preferred_element_type=jnp.float32)
    o10[...] = _ilv(p[:, :DH])
    o11[...] = _ilv(p[:, DH:])


def _edge_proj(edge_attr, We0, We1):
    BE = 2048
    grid = (E_PAD // BE,)
    out = jax.ShapeDtypeStruct((E_PAD, DH // 2), jnp.int32)
    return pl.pallas_call(
        _edge_proj_body,
        grid=grid,
        in_specs=[
            pl.BlockSpec((BE, DE), lambda i: (i, 0)),
            pl.BlockSpec((DE, D), lambda i: (0, 0)),
            pl.BlockSpec((DE, D), lambda i: (0, 0)),
        ],
        out_specs=[pl.BlockSpec((BE, DH // 2), lambda i: (i, 0))] * 4,
        out_shape=[out, out, out, out],
    )(edge_attr, We0, We1)


def _sc_layer_body(x0, x1, ep0, ep1, src, dst, zeros, out0, out1,
                   sidx0, sidx1, didx0, didx1, didx2, didx3,
                   epb0, epb1, xsb0, xsb1, mb0, mb1,
                   semin0, semin1, semg0, semg1, semsc0, semsc1, agg):
    c = lax.axis_index("c")
    s = lax.axis_index("s")
    sidx = [sidx0, sidx1]
    didx = [didx0, didx1, didx2, didx3]
    epb = [epb0, epb1]
    xsb = [xsb0, xsb1]
    mb = [mb0, mb1]
    semin = [semin0, semin1]
    semg = [semg0, semg1]
    semsc = [semsc0, semsc1]

    pltpu.sync_copy(zeros, agg.at[pl.ds(s * ROWS_PER_TILE, ROWS_PER_TILE)])
    plsc.subcore_barrier()

    def do_edges(xh, eph):

        def inload(t, q, b):
            off = (s + t * N_TILES) * CHUNK
            pltpu.make_async_copy(src.at[pl.ds(off, CHUNK)],
                                  sidx[b].at[0], semin[b]).start()
            pltpu.make_async_copy(dst.at[pl.ds(off, CHUNK)],
                                  didx[q].at[0], semin[b]).start()
            pltpu.make_async_copy(eph.at[pl.ds(off, CHUNK)],
                                  epb[b], semin[b]).start()

        def wait_in(b):
            pltpu.make_async_copy(src.at[pl.ds(0, CHUNK)],
                                  sidx[b].at[0], semin[b]).wait()
            pltpu.make_async_copy(dst.at[pl.ds(0, CHUNK)],
                                  didx[0].at[0], semin[b]).wait()
            pltpu.make_async_copy(eph.at[pl.ds(0, CHUNK)],
                                  epb[b], semin[b]).wait()

        def gather(b):
            pltpu.make_async_copy(xh.at[sidx[b].at[0]], xsb[b], semg[b]).start()

        def wait_gather(b):
            pltpu.make_async_copy(xh.at[sidx[b].at[0]], xsb[b], semg[b]).wait()

        def compute(b):
            @plsc.parallel_loop(0, CHUNK, unroll=2)
            def crow(r):
                himask = jnp.int32(-65536)
                for k in range(DH // 32):
                    we = epb[b][r, pl.ds(k * 16, 16)]
                    ea_ = lax.bitcast_convert_type(we << 16, jnp.float32)
                    ec = lax.bitcast_convert_type(we & himask, jnp.float32)
                    sla = pl.ds(k * 16, 16)
                    slc = pl.ds(64 + k * 16, 16)
                    mb[b][r, sla] = jnp.maximum(xsb[b][r, sla] + ea_, 0.0)
                    mb[b][r, slc] = jnp.maximum(xsb[b][r, slc] + ec, 0.0)

        def scatter(q, b):
            pltpu.make_async_copy(mb[b], agg.at[didx[q].at[0]],
                                  semsc[b]).start(add=True)

        def wait_scatter(b):
            pltpu.make_async_copy(mb[b], agg.at[didx[0].at[0]],
                                  semsc[b]).wait()

        def slot(t, q, b, first, has_next, has_next2):
            wait_gather(b)
            if has_next:
                wait_in(1 - b)
                gather(1 - b)
            if not first:
                wait_scatter(b)
            compute(b)
            scatter(q, b)
            if has_next2:
                inload(t + 2, (q + 2) % 4, b)

        inload(0, 0, 0)
        inload(1, 1, 1)
        wait_in(0)
        gather(0)
        slot(0, 0, 0, True, True, True)
        slot(1, 1, 1, True, True, True)
        slot(2, 2, 0, False, True, True)
        slot(3, 3, 1, False, True, True)

        def body(i, carry):
            t0 = 4 * i
            slot(t0 + 0, 0, 0, False, True, True)
            slot(t0 + 1, 1, 1, False, True, True)
            slot(t0 + 2, 2, 0, False, True, True)
            slot(t0 + 3, 3, 1, False, True, True)
            return carry

        lax.fori_loop(1, CPT // 4 - 1, body, 0)

        t0 = CPT - 4
        slot(t0 + 0, 0, 0, False, True, True)
        slot(t0 + 1, 1, 1, False, True, True)
        slot(t0 + 2, 2, 0, False, True, False)
        slot(t0 + 3, 3, 1, False, False, False)
        wait_scatter(0)
        wait_scatter(1)

    @pl.when(c == 0)
    def _():
        do_edges(x0, ep0)

    @pl.when(c == 1)
    def _():
        do_edges(x1, ep1)

    plsc.subcore_barrier()
    rows = pl.ds(s * ROWS_PER_TILE, ROWS_PER_TILE)

    @pl.when(c == 0)
    def _():
        pltpu.sync_copy(agg.at[rows], out0.at[rows])

    @pl.when(c == 1)
    def _():
        pltpu.sync_copy(agg.at[rows], out1.at[rows])


def _sc_layer(x0, x1, ep0, ep1, src, dst, zeros):
    mesh = plsc.VectorSubcoreMesh(core_axis_name="c", subcore_axis_name="s")
    out = jax.ShapeDtypeStruct((N_PAD, DH), jnp.float32)
    idx_t = pltpu.VMEM((1, CHUNK), jnp.int32)
    buf_t = pltpu.VMEM((CHUNK, DH), jnp.float32)
    bufh_t = pltpu.VMEM((CHUNK, DH // 2), jnp.int32)
    f = pl.kernel(
        _sc_layer_body,
        out_type=[out, out],
        mesh=mesh,
        scratch_types=[
            idx_t, idx_t,
            idx_t, idx_t, idx_t, idx_t,
            bufh_t, bufh_t,
            buf_t, buf_t,
            buf_t, buf_t,
            pltpu.SemaphoreType.DMA,
            pltpu.SemaphoreType.DMA,
            pltpu.SemaphoreType.DMA,
            pltpu.SemaphoreType.DMA,
            pltpu.SemaphoreType.DMA,
            pltpu.SemaphoreType.DMA,
            pltpu.VMEM_SHARED((N_PAD, DH), jnp.float32),
        ],
    )
    return f(x0, x1, ep0, ep1, src, dst, zeros)


def _node_update_body(a0, a1, x0, x1, w_ref, b_ref, h0, h1):
    u0 = a0[...] + x0[...]
    u1 = a1[...] + x1[...]
    w = w_ref[...]
    acc = jnp.dot(u0, w[:DH, :], preferred_element_type=jnp.float32)
    acc = acc + jnp.dot(u1, w[DH:, :], preferred_element_type=jnp.float32)
    h = jnp.maximum(acc + b_ref[...], 0.0)
    h0[...] = h[:, :DH]
    h1[...] = h[:, DH:]


def _node_update(agg0, agg1, x0, x1, W, b):
    BN = 2000
    grid = (N // BN,)
    out = jax.ShapeDtypeStruct((N, DH), jnp.float32)
    half = pl.BlockSpec((BN, DH), lambda i: (i, 0))
    return pl.pallas_call(
        _node_update_body,
        grid=grid,
        in_specs=[half, half, half, half,
                  pl.BlockSpec((D, D), lambda i: (0, 0)),
                  pl.BlockSpec((1, D), lambda i: (0, 0))],
        out_specs=[half, half],
        out_shape=[out, out],
    )(agg0, agg1, x0, x1, W, b)


def _nu2_pool_body(batch_ref, a0, a1, x0, x1, w_ref, b_ref,
                   wl0, bl0, wl1, bl1, wemb, bemb, wout, bout, out_ref, acc):
    i = pl.program_id(0)

    @pl.when(i == 0)
    def _():
        acc[...] = jnp.zeros_like(acc)

    u0 = a0[...] + x0[...]
    u1 = a1[...] + x1[...]
    w = w_ref[...]
    hacc = jnp.dot(u0, w[:DH, :], preferred_element_type=jnp.float32)
    hacc = hacc + jnp.dot(u1, w[DH:, :], preferred_element_type=jnp.float32)
    h = jnp.maximum(hacc + b_ref[...], 0.0)

    bi = batch_ref[0, 0, :]
    gid = lax.broadcasted_iota(jnp.int32, (G, bi.shape[0]), 0)
    mask = (gid == bi[None, :]).astype(jnp.float32)
    acc[...] += jnp.dot(mask, h, preferred_element_type=jnp.float32)

    @pl.when(i == pl.num_programs(0) - 1)
    def _():
        p = acc[...]
        a = jnp.maximum(jnp.dot(p, wl0[...], preferred_element_type=jnp.float32) + bl0[...], 0.0)
        a = jnp.maximum(jnp.dot(a, wl1[...], preferred_element_type=jnp.float32) + bl1[...], 0.0)
        e = jnp.dot(a, wemb[...], preferred_element_type=jnp.float32) + bemb[...]
        out_ref[...] = jnp.dot(e, wout[...], preferred_element_type=jnp.float32) + bout[...]


def _nu2_pool(batch3d, agg0, agg1, h0, h1, W, b,
              Wl0, bl0, Wl1, bl1, Wemb, bemb, Wout_p, bout_p):
    BN = 2000
    grid = (N // BN,)
    half = pl.BlockSpec((BN, DH), lambda i: (i, 0))
    full = lambda r, c: pl.BlockSpec((r, c), lambda i: (0, 0))
    return pl.pallas_call(
        _nu2_pool_body,
        grid=grid,
        in_specs=[pl.BlockSpec((1, 1, BN), lambda i: (i, 0, 0)),
                  half, half, half, half,
                  full(D, D), full(1, D),
                  full(D, H), full(1, H),
                  full(H, H), full(1, H),
                  full(H, H), full(1, H),
                  full(H, 128), full(1, 128)],
        out_specs=pl.BlockSpec((G, 128), lambda i: (0, 0)),
        out_shape=jax.ShapeDtypeStruct((G, 128), jnp.float32),
        scratch_shapes=[pltpu.VMEM((G, D), jnp.float32)],
    )(batch3d, agg0, agg1, h0, h1, W, b,
      Wl0, bl0, Wl1, bl1, Wemb, bemb, Wout_p, bout_p)


def kernel(x, edge_index, edge_attr, batch, We0, W0, b0, We1, W1, b1,
           Wl0, bl0, Wl1, bl1, Wemb, bemb, Wout, bout):
    pad = E_PAD - E
    src = jnp.concatenate([edge_index[0], jnp.zeros((pad,), jnp.int32)])
    dst = jnp.concatenate([edge_index[1], jnp.full((pad,), N, jnp.int32)])
    ea = jnp.pad(edge_attr, ((0, pad), (0, 0)))
    x0 = x[:, :DH]
    x1 = x[:, DH:]
    zeros = jnp.zeros((ROWS_PER_TILE, DH), jnp.float32)

    ep00, ep01, ep10, ep11 = _edge_proj(ea, We0, We1)

    a0, a1 = _sc_layer(x0, x1, ep00, ep01, src, dst, zeros)
    h0, h1 = _node_update(a0, a1, x0, x1, W0, b0.reshape(1, D))

    a0, a1 = _sc_layer(h0, h1, ep10, ep11, src, dst, zeros)
    Wout_p = jnp.pad(Wout, ((0, 0), (0, 128 - T)))
    bout_p = jnp.pad(bout, (0, 128 - T)).reshape(1, 128)
    out = _nu2_pool(batch.reshape(N // 2000, 1, 2000), a0, a1, h0, h1,
                    W1, b1.reshape(1, D),
                    Wl0, bl0.reshape(1, H), Wl1, bl1.reshape(1, H),
                    Wemb, bemb.reshape(1, H), Wout_p, bout_p)
    return out[:, :T]


# speedup vs baseline: 1.0412x; 1.0412x over previous
import functools

import jax
import jax.numpy as jnp
from jax import lax
from jax.experimental import pallas as pl
from jax.experimental.pallas import tpu as pltpu
from jax.experimental.pallas import tpu_sc as plsc

N = 10000
E = 160000
D = 256
DE = 16
H = 512
G = 64
T = 10

DH = D // 2
N_PAD = 10112
ROWS_PER_TILE = N_PAD // 16
CHUNK = 64
E_PAD = 163840
N_CHUNKS = E_PAD // CHUNK
N_TILES = 16
CPT = N_CHUNKS // N_TILES


def _ilv(u):
    lb = lax.bitcast_convert_type(u[:, :64], jnp.int32)
    hb = lax.bitcast_convert_type(u[:, 64:], jnp.int32)
    lr = lb + 0x7FFF + ((lb >> 16) & 1)
    hr = hb + 0x7FFF + ((hb >> 16) & 1)
    return (hr & jnp.int32(-65536)) | ((lr >> 16) & 0xFFFF)


def _edge_proj_body(ea_ref, we_ref, o0, o1):
    p = jnp.dot(ea_ref[...], we_ref[...], preferred_element_type=jnp.float32)
    o0[...] = _ilv(p[:, :DH])
    o1[...] = _ilv(p[:, DH:])


def _edge_proj(edge_attr, We):
    BE = 2048
    grid = (E_PAD // BE,)
    out = jax.ShapeDtypeStruct((E_PAD, DH // 2), jnp.int32)
    return pl.pallas_call(
        _edge_proj_body,
        grid=grid,
        in_specs=[
            pl.BlockSpec((BE, DE), lambda i: (i, 0)),
            pl.BlockSpec((DE, D), lambda i: (0, 0)),
        ],
        out_specs=[pl.BlockSpec((BE, DH // 2), lambda i: (i, 0))] * 2,
        out_shape=[out, out],
    )(edge_attr, We)


def _sc_layer_body(x0, x1, ep0, ep1, src, dst, zeros, out0, out1,
                   sidx0, sidx1, didx0, didx1, didx2, didx3,
                   epb0, epb1, xsb0, xsb1, mb0, mb1,
                   semin0, semin1, semg0, semg1, semsc0, semsc1, agg):
    c = lax.axis_index("c")
    s = lax.axis_index("s")
    sidx = [sidx0, sidx1]
    didx = [didx0, didx1, didx2, didx3]
    epb = [epb0, epb1]
    xsb = [xsb0, xsb1]
    mb = [mb0, mb1]
    semin = [semin0, semin1]
    semg = [semg0, semg1]
    semsc = [semsc0, semsc1]

    pltpu.sync_copy(zeros, agg.at[pl.ds(s * ROWS_PER_TILE, ROWS_PER_TILE)])
    plsc.subcore_barrier()

    def do_edges(xh, eph):

        def inload(t, q, b):
            off = (s + t * N_TILES) * CHUNK
            pltpu.make_async_copy(src.at[pl.ds(off, CHUNK)],
                                  sidx[b].at[0], semin[b]).start()
            pltpu.make_async_copy(dst.at[pl.ds(off, CHUNK)],
                                  didx[q].at[0], semin[b]).start()
            pltpu.make_async_copy(eph.at[pl.ds(off, CHUNK)],
                                  epb[b], semin[b]).start()

        def wait_in(b):
            pltpu.make_async_copy(src.at[pl.ds(0, CHUNK)],
                                  sidx[b].at[0], semin[b]).wait()
            pltpu.make_async_copy(dst.at[pl.ds(0, CHUNK)],
                                  didx[0].at[0], semin[b]).wait()
            pltpu.make_async_copy(eph.at[pl.ds(0, CHUNK)],
                                  epb[b], semin[b]).wait()

        def gather(b):
            pltpu.make_async_copy(xh.at[sidx[b].at[0]], xsb[b], semg[b]).start()

        def wait_gather(b):
            pltpu.make_async_copy(xh.at[sidx[b].at[0]], xsb[b], semg[b]).wait()

        def compute(b):
            @plsc.parallel_loop(0, CHUNK, unroll=2)
            def crow(r):
                himask = jnp.int32(-65536)
                for k in range(DH // 32):
                    we = epb[b][r, pl.ds(k * 16, 16)]
                    ea_ = lax.bitcast_convert_type(we << 16, jnp.float32)
                    ec = lax.bitcast_convert_type(we & himask, jnp.float32)
                    sla = pl.ds(k * 16, 16)
                    slc = pl.ds(64 + k * 16, 16)
                    mb[b][r, sla] = jnp.maximum(xsb[b][r, sla] + ea_, 0.0)
                    mb[b][r, slc] = jnp.maximum(xsb[b][r, slc] + ec, 0.0)

        def scatter(q, b):
            pltpu.make_async_copy(mb[b], agg.at[didx[q].at[0]],
                                  semsc[b]).start(add=True)

        def wait_scatter(b):
            pltpu.make_async_copy(mb[b], agg.at[didx[0].at[0]],
                                  semsc[b]).wait()

        def slot(t, q, b, first, has_next, has_next2):
            wait_gather(b)
            if has_next:
                wait_in(1 - b)
                gather(1 - b)
            if not first:
                wait_scatter(b)
            compute(b)
            scatter(q, b)
            if has_next2:
                inload(t + 2, (q + 2) % 4, b)

        inload(0, 0, 0)
        inload(1, 1, 1)
        wait_in(0)
        gather(0)
        slot(0, 0, 0, True, True, True)
        slot(1, 1, 1, True, True, True)
        slot(2, 2, 0, False, True, True)
        slot(3, 3, 1, False, True, True)

        def body(i, carry):
            t0 = 4 * i
            slot(t0 + 0, 0, 0, False, True, True)
            slot(t0 + 1, 1, 1, False, True, True)
            slot(t0 + 2, 2, 0, False, True, True)
            slot(t0 + 3, 3, 1, False, True, True)
            return carry

        lax.fori_loop(1, CPT // 4 - 1, body, 0)

        t0 = CPT - 4
        slot(t0 + 0, 0, 0, False, True, True)
        slot(t0 + 1, 1, 1, False, True, True)
        slot(t0 + 2, 2, 0, False, True, False)
        slot(t0 + 3, 3, 1, False, False, False)
        wait_scatter(0)
        wait_scatter(1)

    @pl.when(c == 0)
    def _():
        do_edges(x0, ep0)

    @pl.when(c == 1)
    def _():
        do_edges(x1, ep1)

    plsc.subcore_barrier()
    rows = pl.ds(s * ROWS_PER_TILE, ROWS_PER_TILE)

    @pl.when(c == 0)
    def _():
        pltpu.sync_copy(agg.at[rows], out0.at[rows])

    @pl.when(c == 1)
    def _():
        pltpu.sync_copy(agg.at[rows], out1.at[rows])


def _sc_layer(x0, x1, ep0, ep1, src, dst, zeros):
    mesh = plsc.VectorSubcoreMesh(core_axis_name="c", subcore_axis_name="s")
    out = jax.ShapeDtypeStruct((N_PAD, DH), jnp.float32)
    idx_t = pltpu.VMEM((1, CHUNK), jnp.int32)
    buf_t = pltpu.VMEM((CHUNK, DH), jnp.float32)
    bufh_t = pltpu.VMEM((CHUNK, DH // 2), jnp.int32)
    f = pl.kernel(
        _sc_layer_body,
        out_type=[out, out],
        mesh=mesh,
        scratch_types=[
            idx_t, idx_t,
            idx_t, idx_t, idx_t, idx_t,
            bufh_t, bufh_t,
            buf_t, buf_t,
            buf_t, buf_t,
            pltpu.SemaphoreType.DMA,
            pltpu.SemaphoreType.DMA,
            pltpu.SemaphoreType.DMA,
            pltpu.SemaphoreType.DMA,
            pltpu.SemaphoreType.DMA,
            pltpu.SemaphoreType.DMA,
            pltpu.VMEM_SHARED((N_PAD, DH), jnp.float32),
        ],
    )
    return f(x0, x1, ep0, ep1, src, dst, zeros)


def _node_update_body(a0, a1, x0, x1, w_ref, b_ref, h0, h1):
    u0 = a0[...] + x0[...]
    u1 = a1[...] + x1[...]
    w = w_ref[...]
    acc = jnp.dot(u0, w[:DH, :], preferred_element_type=jnp.float32)
    acc = acc + jnp.dot(u1, w[DH:, :], preferred_element_type=jnp.float32)
    h = jnp.maximum(acc + b_ref[...], 0.0)
    h0[...] = h[:, :DH]
    h1[...] = h[:, DH:]


def _node_update(agg0, agg1, x0, x1, W, b):
    BN = 2000
    grid = (N // BN,)
    out = jax.ShapeDtypeStruct((N, DH), jnp.float32)
    half = pl.BlockSpec((BN, DH), lambda i: (i, 0))
    return pl.pallas_call(
        _node_update_body,
        grid=grid,
        in_specs=[half, half, half, half,
                  pl.BlockSpec((D, D), lambda i: (0, 0)),
                  pl.BlockSpec((1, D), lambda i: (0, 0))],
        out_specs=[half, half],
        out_shape=[out, out],
    )(agg0, agg1, x0, x1, W, b)


def _nu2_pool_body(batch_ref, a0, a1, x0, x1, w_ref, b_ref,
                   wl0, bl0, wl1, bl1, wemb, bemb, wout, bout, out_ref, acc):
    i = pl.program_id(0)

    @pl.when(i == 0)
    def _():
        acc[...] = jnp.zeros_like(acc)

    u0 = a0[...] + x0[...]
    u1 = a1[...] + x1[...]
    w = w_ref[...]
    hacc = jnp.dot(u0, w[:DH, :], preferred_element_type=jnp.float32)
    hacc = hacc + jnp.dot(u1, w[DH:, :], preferred_element_type=jnp.float32)
    h = jnp.maximum(hacc + b_ref[...], 0.0)

    bi = batch_ref[0, 0, :]
    gid = lax.broadcasted_iota(jnp.int32, (G, bi.shape[0]), 0)
    mask = (gid == bi[None, :]).astype(jnp.float32)
    acc[...] += jnp.dot(mask, h, preferred_element_type=jnp.float32)

    @pl.when(i == pl.num_programs(0) - 1)
    def _():
        p = acc[...]
        a = jnp.maximum(jnp.dot(p, wl0[...], preferred_element_type=jnp.float32) + bl0[...], 0.0)
        a = jnp.maximum(jnp.dot(a, wl1[...], preferred_element_type=jnp.float32) + bl1[...], 0.0)
        e = jnp.dot(a, wemb[...], preferred_element_type=jnp.float32) + bemb[...]
        out_ref[...] = jnp.dot(e, wout[...], preferred_element_type=jnp.float32) + bout[...]


def _nu2_pool(batch3d, agg0, agg1, h0, h1, W, b,
              Wl0, bl0, Wl1, bl1, Wemb, bemb, Wout_p, bout_p):
    BN = 2000
    grid = (N // BN,)
    half = pl.BlockSpec((BN, DH), lambda i: (i, 0))
    full = lambda r, c: pl.BlockSpec((r, c), lambda i: (0, 0))
    return pl.pallas_call(
        _nu2_pool_body,
        grid=grid,
        in_specs=[pl.BlockSpec((1, 1, BN), lambda i: (i, 0, 0)),
                  half, half, half, half,
                  full(D, D), full(1, D),
                  full(D, H), full(1, H),
                  full(H, H), full(1, H),
                  full(H, H), full(1, H),
                  full(H, 128), full(1, 128)],
        out_specs=pl.BlockSpec((G, 128), lambda i: (0, 0)),
        out_shape=jax.ShapeDtypeStruct((G, 128), jnp.float32),
        scratch_shapes=[pltpu.VMEM((G, D), jnp.float32)],
    )(batch3d, agg0, agg1, h0, h1, W, b,
      Wl0, bl0, Wl1, bl1, Wemb, bemb, Wout_p, bout_p)


def kernel(x, edge_index, edge_attr, batch, We0, W0, b0, We1, W1, b1,
           Wl0, bl0, Wl1, bl1, Wemb, bemb, Wout, bout):
    pad = E_PAD - E
    src = jnp.concatenate([edge_index[0], jnp.zeros((pad,), jnp.int32)])
    dst = jnp.concatenate([edge_index[1], jnp.full((pad,), N, jnp.int32)])
    ea = jnp.pad(edge_attr, ((0, pad), (0, 0)))
    x0 = x[:, :DH]
    x1 = x[:, DH:]
    zeros = jnp.zeros((ROWS_PER_TILE, DH), jnp.float32)

    ep00, ep01 = _edge_proj(ea, We0)

    a0, a1 = _sc_layer(x0, x1, ep00, ep01, src, dst, zeros)
    ep10, ep11 = _edge_proj(ea, We1)
    h0, h1 = _node_update(a0, a1, x0, x1, W0, b0.reshape(1, D))

    a0, a1 = _sc_layer(h0, h1, ep10, ep11, src, dst, zeros)
    Wout_p = jnp.pad(Wout, ((0, 0), (0, 128 - T)))
    bout_p = jnp.pad(bout, (0, 128 - T)).reshape(1, 128)
    out = _nu2_pool(batch.reshape(N // 2000, 1, 2000), a0, a1, h0, h1,
                    W1, b1.reshape(1, D),
                    Wl0, bl0.reshape(1, H), Wl1, bl1.reshape(1, H),
                    Wemb, bemb.reshape(1, H), Wout_p, bout_p)
    return out[:, :T]


# speedup vs baseline: 1.0429x; 1.0016x over previous
import functools

import jax
import jax.numpy as jnp
from jax import lax
from jax.experimental import pallas as pl
from jax.experimental.pallas import tpu as pltpu
from jax.experimental.pallas import tpu_sc as plsc

N = 10000
E = 160000
D = 256
DE = 16
H = 512
G = 64
T = 10

DH = D // 2
N_PAD = 10112
ROWS_PER_TILE = N_PAD // 16
CHUNK = 64
E_PAD = 163840
N_CHUNKS = E_PAD // CHUNK
N_TILES = 16
CPT = N_CHUNKS // N_TILES


def _ilv(u):
    lb = lax.bitcast_convert_type(u[:, :64], jnp.int32)
    hb = lax.bitcast_convert_type(u[:, 64:], jnp.int32)
    lr = lb + 0x7FFF + ((lb >> 16) & 1)
    hr = hb + 0x7FFF + ((hb >> 16) & 1)
    return (hr & jnp.int32(-65536)) | ((lr >> 16) & 0xFFFF)


def _edge_proj_body(ea_ref, we_ref, o0, o1):
    p = jnp.dot(ea_ref[...], we_ref[...], preferred_element_type=jnp.float32)
    o0[...] = _ilv(p[:, :DH])
    o1[...] = _ilv(p[:, DH:])


def _edge_proj(edge_attr, We):
    BE = 2048
    grid = (E_PAD // BE,)
    out = jax.ShapeDtypeStruct((E_PAD, DH // 2), jnp.int32)
    return pl.pallas_call(
        _edge_proj_body,
        grid=grid,
        in_specs=[
            pl.BlockSpec((BE, DE), lambda i: (i, 0)),
            pl.BlockSpec((DE, D), lambda i: (0, 0)),
        ],
        out_specs=[pl.BlockSpec((BE, DH // 2), lambda i: (i, 0))] * 2,
        out_shape=[out, out],
    )(edge_attr, We)


def _sc_layer_body(x0, x1, ep0, ep1, src, dst, zeros, out0, out1,
                   sidx0, sidx1, didx0, didx1, didx2, didx3,
                   epb0, epb1, xsb0, xsb1, mb0, mb1,
                   semin0, semin1, semg0, semg1, semsc0, semsc1, agg):
    c = lax.axis_index("c")
    s = lax.axis_index("s")
    sidx = [sidx0, sidx1]
    didx = [didx0, didx1, didx2, didx3]
    epb = [epb0, epb1]
    xsb = [xsb0, xsb1]
    mb = [mb0, mb1]
    semin = [semin0, semin1]
    semg = [semg0, semg1]
    semsc = [semsc0, semsc1]

    pltpu.sync_copy(zeros, agg.at[pl.ds(s * ROWS_PER_TILE, ROWS_PER_TILE)])
    plsc.subcore_barrier()

    def do_edges(xh, eph):

        def inload(t, q, b):
            off = (s + t * N_TILES) * CHUNK
            pltpu.make_async_copy(src.at[pl.ds(off, CHUNK)],
                                  sidx[b].at[0], semin[b]).start()
            pltpu.make_async_copy(dst.at[pl.ds(off, CHUNK)],
                                  didx[q].at[0], semin[b]).start()
            pltpu.make_async_copy(eph.at[pl.ds(off, CHUNK)],
                                  epb[b], semin[b]).start()

        def wait_in(b):
            pltpu.make_async_copy(src.at[pl.ds(0, CHUNK)],
                                  sidx[b].at[0], semin[b]).wait()
            pltpu.make_async_copy(dst.at[pl.ds(0, CHUNK)],
                                  didx[0].at[0], semin[b]).wait()
            pltpu.make_async_copy(eph.at[pl.ds(0, CHUNK)],
                                  epb[b], semin[b]).wait()

        def gather(b):
            pltpu.make_async_copy(xh.at[sidx[b].at[0]], xsb[b], semg[b]).start()

        def wait_gather(b):
            pltpu.make_async_copy(xh.at[sidx[b].at[0]], xsb[b], semg[b]).wait()

        def compute(b):
            @plsc.parallel_loop(0, CHUNK, unroll=4)
            def crow(r):
                himask = jnp.int32(-65536)
                for k in range(DH // 32):
                    we = epb[b][r, pl.ds(k * 16, 16)]
                    ea_ = lax.bitcast_convert_type(we << 16, jnp.float32)
                    ec = lax.bitcast_convert_type(we & himask, jnp.float32)
                    sla = pl.ds(k * 16, 16)
                    slc = pl.ds(64 + k * 16, 16)
                    mb[b][r, sla] = jnp.maximum(xsb[b][r, sla] + ea_, 0.0)
                    mb[b][r, slc] = jnp.maximum(xsb[b][r, slc] + ec, 0.0)

        def scatter(q, b):
            pltpu.make_async_copy(mb[b], agg.at[didx[q].at[0]],
                                  semsc[b]).start(add=True)

        def wait_scatter(b):
            pltpu.make_async_copy(mb[b], agg.at[didx[0].at[0]],
                                  semsc[b]).wait()

        def slot(t, q, b, first, has_next, has_next2):
            wait_gather(b)
            if has_next:
                wait_in(1 - b)
                gather(1 - b)
            if not first:
                wait_scatter(b)
            compute(b)
            scatter(q, b)
            if has_next2:
                inload(t + 2, (q + 2) % 4, b)

        inload(0, 0, 0)
        inload(1, 1, 1)
        wait_in(0)
        gather(0)
        slot(0, 0, 0, True, True, True)
        slot(1, 1, 1, True, True, True)
        slot(2, 2, 0, False, True, True)
        slot(3, 3, 1, False, True, True)

        def body(i, carry):
            t0 = 4 * i
            slot(t0 + 0, 0, 0, False, True, True)
            slot(t0 + 1, 1, 1, False, True, True)
            slot(t0 + 2, 2, 0, False, True, True)
            slot(t0 + 3, 3, 1, False, True, True)
            return carry

        lax.fori_loop(1, CPT // 4 - 1, body, 0)

        t0 = CPT - 4
        slot(t0 + 0, 0, 0, False, True, True)
        slot(t0 + 1, 1, 1, False, True, True)
        slot(t0 + 2, 2, 0, False, True, False)
        slot(t0 + 3, 3, 1, False, False, False)
        wait_scatter(0)
        wait_scatter(1)

    @pl.when(c == 0)
    def _():
        do_edges(x0, ep0)

    @pl.when(c == 1)
    def _():
        do_edges(x1, ep1)

    plsc.subcore_barrier()
    rows = pl.ds(s * ROWS_PER_TILE, ROWS_PER_TILE)

    @pl.when(c == 0)
    def _():
        pltpu.sync_copy(agg.at[rows], out0.at[rows])

    @pl.when(c == 1)
    def _():
        pltpu.sync_copy(agg.at[rows], out1.at[rows])


def _sc_layer(x0, x1, ep0, ep1, src, dst, zeros):
    mesh = plsc.VectorSubcoreMesh(core_axis_name="c", subcore_axis_name="s")
    out = jax.ShapeDtypeStruct((N_PAD, DH), jnp.float32)
    idx_t = pltpu.VMEM((1, CHUNK), jnp.int32)
    buf_t = pltpu.VMEM((CHUNK, DH), jnp.float32)
    bufh_t = pltpu.VMEM((CHUNK, DH // 2), jnp.int32)
    f = pl.kernel(
        _sc_layer_body,
        out_type=[out, out],
        mesh=mesh,
        scratch_types=[
            idx_t, idx_t,
            idx_t, idx_t, idx_t, idx_t,
            bufh_t, bufh_t,
            buf_t, buf_t,
            buf_t, buf_t,
            pltpu.SemaphoreType.DMA,
            pltpu.SemaphoreType.DMA,
            pltpu.SemaphoreType.DMA,
            pltpu.SemaphoreType.DMA,
            pltpu.SemaphoreType.DMA,
            pltpu.SemaphoreType.DMA,
            pltpu.VMEM_SHARED((N_PAD, DH), jnp.float32),
        ],
    )
    return f(x0, x1, ep0, ep1, src, dst, zeros)


def _node_update_body(a0, a1, x0, x1, w_ref, b_ref, h0, h1):
    u0 = a0[...] + x0[...]
    u1 = a1[...] + x1[...]
    w = w_ref[...]
    acc = jnp.dot(u0, w[:DH, :], preferred_element_type=jnp.float32)
    acc = acc + jnp.dot(u1, w[DH:, :], preferred_element_type=jnp.float32)
    h = jnp.maximum(acc + b_ref[...], 0.0)
    h0[...] = h[:, :DH]
    h1[...] = h[:, DH:]


def _node_update(agg0, agg1, x0, x1, W, b):
    BN = 2000
    grid = (N // BN,)
    out = jax.ShapeDtypeStruct((N, DH), jnp.float32)
    half = pl.BlockSpec((BN, DH), lambda i: (i, 0))
    return pl.pallas_call(
        _node_update_body,
        grid=grid,
        in_specs=[half, half, half, half,
                  pl.BlockSpec((D, D), lambda i: (0, 0)),
                  pl.BlockSpec((1, D), lambda i: (0, 0))],
        out_specs=[half, half],
        out_shape=[out, out],
    )(agg0, agg1, x0, x1, W, b)


def _nu2_pool_body(batch_ref, a0, a1, x0, x1, w_ref, b_ref,
                   wl0, bl0, wl1, bl1, wemb, bemb, wout, bout, out_ref, acc):
    i = pl.program_id(0)

    @pl.when(i == 0)
    def _():
        acc[...] = jnp.zeros_like(acc)

    u0 = a0[...] + x0[...]
    u1 = a1[...] + x1[...]
    w = w_ref[...]
    hacc = jnp.dot(u0, w[:DH, :], preferred_element_type=jnp.float32)
    hacc = hacc + jnp.dot(u1, w[DH:, :], preferred_element_type=jnp.float32)
    h = jnp.maximum(hacc + b_ref[...], 0.0)

    bi = batch_ref[0, 0, :]
    gid = lax.broadcasted_iota(jnp.int32, (G, bi.shape[0]), 0)
    mask = (gid == bi[None, :]).astype(jnp.float32)
    acc[...] += jnp.dot(mask, h, preferred_element_type=jnp.float32)

    @pl.when(i == pl.num_programs(0) - 1)
    def _():
        p = acc[...]
        a = jnp.maximum(jnp.dot(p, wl0[...], preferred_element_type=jnp.float32) + bl0[...], 0.0)
        a = jnp.maximum(jnp.dot(a, wl1[...], preferred_element_type=jnp.float32) + bl1[...], 0.0)
        e = jnp.dot(a, wemb[...], preferred_element_type=jnp.float32) + bemb[...]
        out_ref[...] = jnp.dot(e, wout[...], preferred_element_type=jnp.float32) + bout[...]


def _nu2_pool(batch3d, agg0, agg1, h0, h1, W, b,
              Wl0, bl0, Wl1, bl1, Wemb, bemb, Wout_p, bout_p):
    BN = 2000
    grid = (N // BN,)
    half = pl.BlockSpec((BN, DH), lambda i: (i, 0))
    full = lambda r, c: pl.BlockSpec((r, c), lambda i: (0, 0))
    return pl.pallas_call(
        _nu2_pool_body,
        grid=grid,
        in_specs=[pl.BlockSpec((1, 1, BN), lambda i: (i, 0, 0)),
                  half, half, half, half,
                  full(D, D), full(1, D),
                  full(D, H), full(1, H),
                  full(H, H), full(1, H),
                  full(H, H), full(1, H),
                  full(H, 128), full(1, 128)],
        out_specs=pl.BlockSpec((G, 128), lambda i: (0, 0)),
        out_shape=jax.ShapeDtypeStruct((G, 128), jnp.float32),
        scratch_shapes=[pltpu.VMEM((G, D), jnp.float32)],
    )(batch3d, agg0, agg1, h0, h1, W, b,
      Wl0, bl0, Wl1, bl1, Wemb, bemb, Wout_p, bout_p)


def kernel(x, edge_index, edge_attr, batch, We0, W0, b0, We1, W1, b1,
           Wl0, bl0, Wl1, bl1, Wemb, bemb, Wout, bout):
    pad = E_PAD - E
    src = jnp.concatenate([edge_index[0], jnp.zeros((pad,), jnp.int32)])
    dst = jnp.concatenate([edge_index[1], jnp.full((pad,), N, jnp.int32)])
    ea = jnp.pad(edge_attr, ((0, pad), (0, 0)))
    x0 = x[:, :DH]
    x1 = x[:, DH:]
    zeros = jnp.zeros((ROWS_PER_TILE, DH), jnp.float32)

    ep00, ep01 = _edge_proj(ea, We0)

    a0, a1 = _sc_layer(x0, x1, ep00, ep01, src, dst, zeros)
    ep10, ep11 = _edge_proj(ea, We1)
    h0, h1 = _node_update(a0, a1, x0, x1, W0, b0.reshape(1, D))

    a0, a1 = _sc_layer(h0, h1, ep10, ep11, src, dst, zeros)
    Wout_p = jnp.pad(Wout, ((0, 0), (0, 128 - T)))
    bout_p = jnp.pad(bout, (0, 128 - T)).reshape(1, 128)
    out = _nu2_pool(batch.reshape(N // 2000, 1, 2000), a0, a1, h0, h1,
                    W1, b1.reshape(1, D),
                    Wl0, bl0.reshape(1, H), Wl1, bl1.reshape(1, H),
                    Wemb, bemb.reshape(1, H), Wout_p, bout_p)
    return out[:, :T]


# speedup vs baseline: 1.0845x; 1.0399x over previous
import functools

import jax
import jax.numpy as jnp
from jax import lax
from jax.experimental import pallas as pl
from jax.experimental.pallas import tpu as pltpu
from jax.experimental.pallas import tpu_sc as plsc

N = 10000
E = 160000
D = 256
DE = 16
H = 512
G = 64
T = 10

DH = D // 2
N_PAD = 10112
ROWS_PER_TILE = N_PAD // 16
CHUNK = 80
E_PAD = 163840
N_CHUNKS = E_PAD // CHUNK
N_TILES = 16
CPT = N_CHUNKS // N_TILES


def _ilv(u):
    lb = lax.bitcast_convert_type(u[:, :64], jnp.int32)
    hb = lax.bitcast_convert_type(u[:, 64:], jnp.int32)
    lr = lb + 0x7FFF + ((lb >> 16) & 1)
    hr = hb + 0x7FFF + ((hb >> 16) & 1)
    return (hr & jnp.int32(-65536)) | ((lr >> 16) & 0xFFFF)


def _edge_proj_body(ea_ref, we_ref, o0, o1):
    p = jnp.dot(ea_ref[...], we_ref[...], preferred_element_type=jnp.float32)
    o0[...] = _ilv(p[:, :DH])
    o1[...] = _ilv(p[:, DH:])


def _edge_proj(edge_attr, We):
    BE = 2048
    grid = (E_PAD // BE,)
    out = jax.ShapeDtypeStruct((E_PAD, DH // 2), jnp.int32)
    return pl.pallas_call(
        _edge_proj_body,
        grid=grid,
        in_specs=[
            pl.BlockSpec((BE, DE), lambda i: (i, 0)),
            pl.BlockSpec((DE, D), lambda i: (0, 0)),
        ],
        out_specs=[pl.BlockSpec((BE, DH // 2), lambda i: (i, 0))] * 2,
        out_shape=[out, out],
    )(edge_attr, We)


def _sc_layer_body(x0, x1, ep0, ep1, src, dst, zeros, out0, out1,
                   sidx0, sidx1, didx0, didx1, didx2, didx3,
                   epb0, epb1, xsb0, xsb1,
                   semin0, semin1, semg0, semg1, semsc0, semsc1, agg):
    c = lax.axis_index("c")
    s = lax.axis_index("s")
    sidx = [sidx0, sidx1]
    didx = [didx0, didx1, didx2, didx3]
    epb = [epb0, epb1]
    xsb = [xsb0, xsb1]
    semin = [semin0, semin1]
    semg = [semg0, semg1]
    semsc = [semsc0, semsc1]

    pltpu.sync_copy(zeros, agg.at[pl.ds(s * ROWS_PER_TILE, ROWS_PER_TILE)])
    plsc.subcore_barrier()

    def do_edges(xh, eph):

        def inload(t, q, b):
            off = (s + t * N_TILES) * CHUNK
            pltpu.make_async_copy(src.at[pl.ds(off, CHUNK)],
                                  sidx[b].at[0], semin[b]).start()
            pltpu.make_async_copy(dst.at[pl.ds(off, CHUNK)],
                                  didx[q].at[0], semin[b]).start()
            pltpu.make_async_copy(eph.at[pl.ds(off, CHUNK)],
                                  epb[b], semin[b]).start()

        def wait_in(b):
            pltpu.make_async_copy(src.at[pl.ds(0, CHUNK)],
                                  sidx[b].at[0], semin[b]).wait()
            pltpu.make_async_copy(dst.at[pl.ds(0, CHUNK)],
                                  didx[0].at[0], semin[b]).wait()
            pltpu.make_async_copy(eph.at[pl.ds(0, CHUNK)],
                                  epb[b], semin[b]).wait()

        def gather(b):
            pltpu.make_async_copy(xh.at[sidx[b].at[0]], xsb[b], semg[b]).start()

        def wait_gather(b):
            pltpu.make_async_copy(xh.at[sidx[b].at[0]], xsb[b], semg[b]).wait()

        def compute(b):
            @plsc.parallel_loop(0, CHUNK, unroll=4)
            def crow(r):
                himask = jnp.int32(-65536)
                for k in range(DH // 32):
                    we = epb[b][r, pl.ds(k * 16, 16)]
                    ea_ = lax.bitcast_convert_type(we << 16, jnp.float32)
                    ec = lax.bitcast_convert_type(we & himask, jnp.float32)
                    sla = pl.ds(k * 16, 16)
                    slc = pl.ds(64 + k * 16, 16)
                    xsb[b][r, sla] = jnp.maximum(xsb[b][r, sla] + ea_, 0.0)
                    xsb[b][r, slc] = jnp.maximum(xsb[b][r, slc] + ec, 0.0)

        def scatter(q, b):
            pltpu.make_async_copy(xsb[b], agg.at[didx[q].at[0]],
                                  semsc[b]).start(add=True)

        def wait_scatter(b):
            pltpu.make_async_copy(xsb[b], agg.at[didx[0].at[0]],
                                  semsc[b]).wait()

        def slot(t, q, b, first, has_next, has_next2):
            wait_gather(b)
            if has_next:
                wait_in(1 - b)
                if not first:
                    wait_scatter(1 - b)
                gather(1 - b)
            compute(b)
            scatter(q, b)
            if has_next2:
                inload(t + 2, (q + 2) % 4, b)

        inload(0, 0, 0)
        inload(1, 1, 1)
        wait_in(0)
        gather(0)
        slot(0, 0, 0, True, True, True)
        slot(1, 1, 1, False, True, True)
        slot(2, 2, 0, False, True, True)
        slot(3, 3, 1, False, True, True)

        def body(i, carry):
            t0 = 4 * i
            slot(t0 + 0, 0, 0, False, True, True)
            slot(t0 + 1, 1, 1, False, True, True)
            slot(t0 + 2, 2, 0, False, True, True)
            slot(t0 + 3, 3, 1, False, True, True)
            return carry

        lax.fori_loop(1, CPT // 4 - 1, body, 0)

        t0 = CPT - 4
        slot(t0 + 0, 0, 0, False, True, True)
        slot(t0 + 1, 1, 1, False, True, True)
        slot(t0 + 2, 2, 0, False, True, False)
        slot(t0 + 3, 3, 1, False, False, False)
        wait_scatter(0)
        wait_scatter(1)

    @pl.when(c == 0)
    def _():
        do_edges(x0, ep0)

    @pl.when(c == 1)
    def _():
        do_edges(x1, ep1)

    plsc.subcore_barrier()
    rows = pl.ds(s * ROWS_PER_TILE, ROWS_PER_TILE)

    @pl.when(c == 0)
    def _():
        pltpu.sync_copy(agg.at[rows], out0.at[rows])

    @pl.when(c == 1)
    def _():
        pltpu.sync_copy(agg.at[rows], out1.at[rows])


def _sc_layer(x0, x1, ep0, ep1, src, dst, zeros):
    mesh = plsc.VectorSubcoreMesh(core_axis_name="c", subcore_axis_name="s")
    out = jax.ShapeDtypeStruct((N_PAD, DH), jnp.float32)
    idx_t = pltpu.VMEM((1, CHUNK), jnp.int32)
    buf_t = pltpu.VMEM((CHUNK, DH), jnp.float32)
    bufh_t = pltpu.VMEM((CHUNK, DH // 2), jnp.int32)
    f = pl.kernel(
        _sc_layer_body,
        out_type=[out, out],
        mesh=mesh,
        scratch_types=[
            idx_t, idx_t,
            idx_t, idx_t, idx_t, idx_t,
            bufh_t, bufh_t,
            buf_t, buf_t,
            pltpu.SemaphoreType.DMA,
            pltpu.SemaphoreType.DMA,
            pltpu.SemaphoreType.DMA,
            pltpu.SemaphoreType.DMA,
            pltpu.SemaphoreType.DMA,
            pltpu.SemaphoreType.DMA,
            pltpu.VMEM_SHARED((N_PAD, DH), jnp.float32),
        ],
    )
    return f(x0, x1, ep0, ep1, src, dst, zeros)


def _node_update_body(a0, a1, x0, x1, w_ref, b_ref, h0, h1):
    u0 = a0[...] + x0[...]
    u1 = a1[...] + x1[...]
    w = w_ref[...]
    acc = jnp.dot(u0, w[:DH, :], preferred_element_type=jnp.float32)
    acc = acc + jnp.dot(u1, w[DH:, :], preferred_element_type=jnp.float32)
    h = jnp.maximum(acc + b_ref[...], 0.0)
    h0[...] = h[:, :DH]
    h1[...] = h[:, DH:]


def _node_update(agg0, agg1, x0, x1, W, b):
    BN = 2000
    grid = (N // BN,)
    out = jax.ShapeDtypeStruct((N, DH), jnp.float32)
    half = pl.BlockSpec((BN, DH), lambda i: (i, 0))
    return pl.pallas_call(
        _node_update_body,
        grid=grid,
        in_specs=[half, half, half, half,
                  pl.BlockSpec((D, D), lambda i: (0, 0)),
                  pl.BlockSpec((1, D), lambda i: (0, 0))],
        out_specs=[half, half],
        out_shape=[out, out],
    )(agg0, agg1, x0, x1, W, b)


def _nu2_pool_body(batch_ref, a0, a1, x0, x1, w_ref, b_ref,
                   wl0, bl0, wl1, bl1, wemb, bemb, wout, bout, out_ref, acc):
    i = pl.program_id(0)

    @pl.when(i == 0)
    def _():
        acc[...] = jnp.zeros_like(acc)

    u0 = a0[...] + x0[...]
    u1 = a1[...] + x1[...]
    w = w_ref[...]
    hacc = jnp.dot(u0, w[:DH, :], preferred_element_type=jnp.float32)
    hacc = hacc + jnp.dot(u1, w[DH:, :], preferred_element_type=jnp.float32)
    h = jnp.maximum(hacc + b_ref[...], 0.0)

    bi = batch_ref[0, 0, :]
    gid = lax.broadcasted_iota(jnp.int32, (G, bi.shape[0]), 0)
    mask = (gid == bi[None, :]).astype(jnp.float32)
    acc[...] += jnp.dot(mask, h, preferred_element_type=jnp.float32)

    @pl.when(i == pl.num_programs(0) - 1)
    def _():
        p = acc[...]
        a = jnp.maximum(jnp.dot(p, wl0[...], preferred_element_type=jnp.float32) + bl0[...], 0.0)
        a = jnp.maximum(jnp.dot(a, wl1[...], preferred_element_type=jnp.float32) + bl1[...], 0.0)
        e = jnp.dot(a, wemb[...], preferred_element_type=jnp.float32) + bemb[...]
        out_ref[...] = jnp.dot(e, wout[...], preferred_element_type=jnp.float32) + bout[...]


def _nu2_pool(batch3d, agg0, agg1, h0, h1, W, b,
              Wl0, bl0, Wl1, bl1, Wemb, bemb, Wout_p, bout_p):
    BN = 2000
    grid = (N // BN,)
    half = pl.BlockSpec((BN, DH), lambda i: (i, 0))
    full = lambda r, c: pl.BlockSpec((r, c), lambda i: (0, 0))
    return pl.pallas_call(
        _nu2_pool_body,
        grid=grid,
        in_specs=[pl.BlockSpec((1, 1, BN), lambda i: (i, 0, 0)),
                  half, half, half, half,
                  full(D, D), full(1, D),
                  full(D, H), full(1, H),
                  full(H, H), full(1, H),
                  full(H, H), full(1, H),
                  full(H, 128), full(1, 128)],
        out_specs=pl.BlockSpec((G, 128), lambda i: (0, 0)),
        out_shape=jax.ShapeDtypeStruct((G, 128), jnp.float32),
        scratch_shapes=[pltpu.VMEM((G, D), jnp.float32)],
    )(batch3d, agg0, agg1, h0, h1, W, b,
      Wl0, bl0, Wl1, bl1, Wemb, bemb, Wout_p, bout_p)


def kernel(x, edge_index, edge_attr, batch, We0, W0, b0, We1, W1, b1,
           Wl0, bl0, Wl1, bl1, Wemb, bemb, Wout, bout):
    pad = E_PAD - E
    src = jnp.concatenate([edge_index[0], jnp.zeros((pad,), jnp.int32)])
    dst = jnp.concatenate([edge_index[1], jnp.full((pad,), N, jnp.int32)])
    ea = jnp.pad(edge_attr, ((0, pad), (0, 0)))
    x0 = x[:, :DH]
    x1 = x[:, DH:]
    zeros = jnp.zeros((ROWS_PER_TILE, DH), jnp.float32)

    ep00, ep01 = _edge_proj(ea, We0)

    a0, a1 = _sc_layer(x0, x1, ep00, ep01, src, dst, zeros)
    ep10, ep11 = _edge_proj(ea, We1)
    h0, h1 = _node_update(a0, a1, x0, x1, W0, b0.reshape(1, D))

    a0, a1 = _sc_layer(h0, h1, ep10, ep11, src, dst, zeros)
    Wout_p = jnp.pad(Wout, ((0, 0), (0, 128 - T)))
    bout_p = jnp.pad(bout, (0, 128 - T)).reshape(1, 128)
    out = _nu2_pool(batch.reshape(N // 2000, 1, 2000), a0, a1, h0, h1,
                    W1, b1.reshape(1, D),
                    Wl0, bl0.reshape(1, H), Wl1, bl1.reshape(1, H),
                    Wemb, bemb.reshape(1, H), Wout_p, bout_p)
    return out[:, :T]


# speedup vs baseline: 1.0848x; 1.0003x over previous
import functools

import jax
import jax.numpy as jnp
from jax import lax
from jax.experimental import pallas as pl
from jax.experimental.pallas import tpu as pltpu
from jax.experimental.pallas import tpu_sc as plsc

N = 10000
E = 160000
D = 256
DE = 16
H = 512
G = 64
T = 10

DH = D // 2
N_PAD = 10112
ROWS_PER_TILE = N_PAD // 16
CHUNK = 80
E_PAD = 163840
N_CHUNKS = E_PAD // CHUNK
N_TILES = 16
CPT = N_CHUNKS // N_TILES


def _ilv(u):
    lb = lax.bitcast_convert_type(u[:, :64], jnp.int32)
    hb = lax.bitcast_convert_type(u[:, 64:], jnp.int32)
    lr = lb + 0x7FFF + ((lb >> 16) & 1)
    hr = hb + 0x7FFF + ((hb >> 16) & 1)
    return (hr & jnp.int32(-65536)) | ((lr >> 16) & 0xFFFF)


def _edge_proj_body(ea_ref, we_ref, o0, o1):
    p = jnp.dot(ea_ref[...], we_ref[...], preferred_element_type=jnp.float32)
    o0[...] = _ilv(p[:, :DH])
    o1[...] = _ilv(p[:, DH:])


def _edge_proj(edge_attr, We):
    BE = 2048
    grid = (E_PAD // BE,)
    out = jax.ShapeDtypeStruct((E_PAD, DH // 2), jnp.int32)
    return pl.pallas_call(
        _edge_proj_body,
        grid=grid,
        in_specs=[
            pl.BlockSpec((BE, DE), lambda i: (i, 0)),
            pl.BlockSpec((DE, D), lambda i: (0, 0)),
        ],
        out_specs=[pl.BlockSpec((BE, DH // 2), lambda i: (i, 0))] * 2,
        out_shape=[out, out],
    )(edge_attr, We)


def _sc_layer_body(x0, x1, ep0, ep1, src, dst, zeros, out0, out1,
                   sidx0, sidx1, didx0, didx1, didx2, didx3,
                   epb0, epb1, xsb0, xsb1,
                   semin0, semin1, semg0, semg1, semsc0, semsc1, agg):
    c = lax.axis_index("c")
    s = lax.axis_index("s")
    sidx = [sidx0, sidx1]
    didx = [didx0, didx1, didx2, didx3]
    epb = [epb0, epb1]
    xsb = [xsb0, xsb1]
    semin = [semin0, semin1]
    semg = [semg0, semg1]
    semsc = [semsc0, semsc1]

    def do_edges(xh, eph):

        def inload(t, q, b):
            off = (s + t * N_TILES) * CHUNK
            pltpu.make_async_copy(src.at[pl.ds(off, CHUNK)],
                                  sidx[b].at[0], semin[b]).start()
            pltpu.make_async_copy(dst.at[pl.ds(off, CHUNK)],
                                  didx[q].at[0], semin[b]).start()
            pltpu.make_async_copy(eph.at[pl.ds(off, CHUNK)],
                                  epb[b], semin[b]).start()

        def wait_in(b):
            pltpu.make_async_copy(src.at[pl.ds(0, CHUNK)],
                                  sidx[b].at[0], semin[b]).wait()
            pltpu.make_async_copy(dst.at[pl.ds(0, CHUNK)],
                                  didx[0].at[0], semin[b]).wait()
            pltpu.make_async_copy(eph.at[pl.ds(0, CHUNK)],
                                  epb[b], semin[b]).wait()

        def gather(b):
            pltpu.make_async_copy(xh.at[sidx[b].at[0]], xsb[b], semg[b]).start()

        def wait_gather(b):
            pltpu.make_async_copy(xh.at[sidx[b].at[0]], xsb[b], semg[b]).wait()

        def compute(b):
            @plsc.parallel_loop(0, CHUNK, unroll=4)
            def crow(r):
                himask = jnp.int32(-65536)
                for k in range(DH // 32):
                    we = epb[b][r, pl.ds(k * 16, 16)]
                    ea_ = lax.bitcast_convert_type(we << 16, jnp.float32)
                    ec = lax.bitcast_convert_type(we & himask, jnp.float32)
                    sla = pl.ds(k * 16, 16)
                    slc = pl.ds(64 + k * 16, 16)
                    xsb[b][r, sla] = jnp.maximum(xsb[b][r, sla] + ea_, 0.0)
                    xsb[b][r, slc] = jnp.maximum(xsb[b][r, slc] + ec, 0.0)

        def scatter(q, b):
            pltpu.make_async_copy(xsb[b], agg.at[didx[q].at[0]],
                                  semsc[b]).start(add=True)

        def wait_scatter(b):
            pltpu.make_async_copy(xsb[b], agg.at[didx[0].at[0]],
                                  semsc[b]).wait()

        def slot(t, q, b, first, has_next, has_next2):
            wait_gather(b)
            if has_next:
                wait_in(1 - b)
                if not first:
                    wait_scatter(1 - b)
                gather(1 - b)
            compute(b)
            scatter(q, b)
            if has_next2:
                inload(t + 2, (q + 2) % 4, b)

        inload(0, 0, 0)
        inload(1, 1, 1)
        pltpu.sync_copy(zeros, agg.at[pl.ds(s * ROWS_PER_TILE, ROWS_PER_TILE)])
        plsc.subcore_barrier()
        wait_in(0)
        gather(0)
        slot(0, 0, 0, True, True, True)
        slot(1, 1, 1, False, True, True)
        slot(2, 2, 0, False, True, True)
        slot(3, 3, 1, False, True, True)

        def body(i, carry):
            t0 = 4 * i
            slot(t0 + 0, 0, 0, False, True, True)
            slot(t0 + 1, 1, 1, False, True, True)
            slot(t0 + 2, 2, 0, False, True, True)
            slot(t0 + 3, 3, 1, False, True, True)
            return carry

        lax.fori_loop(1, CPT // 4 - 1, body, 0)

        t0 = CPT - 4
        slot(t0 + 0, 0, 0, False, True, True)
        slot(t0 + 1, 1, 1, False, True, True)
        slot(t0 + 2, 2, 0, False, True, False)
        slot(t0 + 3, 3, 1, False, False, False)
        wait_scatter(0)
        wait_scatter(1)

    @pl.when(c == 0)
    def _():
        do_edges(x0, ep0)

    @pl.when(c == 1)
    def _():
        do_edges(x1, ep1)

    plsc.subcore_barrier()
    rows = pl.ds(s * ROWS_PER_TILE, ROWS_PER_TILE)

    @pl.when(c == 0)
    def _():
        pltpu.sync_copy(agg.at[rows], out0.at[rows])

    @pl.when(c == 1)
    def _():
        pltpu.sync_copy(agg.at[rows], out1.at[rows])


def _sc_layer(x0, x1, ep0, ep1, src, dst, zeros):
    mesh = plsc.VectorSubcoreMesh(core_axis_name="c", subcore_axis_name="s")
    out = jax.ShapeDtypeStruct((N_PAD, DH), jnp.float32)
    idx_t = pltpu.VMEM((1, CHUNK), jnp.int32)
    buf_t = pltpu.VMEM((CHUNK, DH), jnp.float32)
    bufh_t = pltpu.VMEM((CHUNK, DH // 2), jnp.int32)
    f = pl.kernel(
        _sc_layer_body,
        out_type=[out, out],
        mesh=mesh,
        scratch_types=[
            idx_t, idx_t,
            idx_t, idx_t, idx_t, idx_t,
            bufh_t, bufh_t,
            buf_t, buf_t,
            pltpu.SemaphoreType.DMA,
            pltpu.SemaphoreType.DMA,
            pltpu.SemaphoreType.DMA,
            pltpu.SemaphoreType.DMA,
            pltpu.SemaphoreType.DMA,
            pltpu.SemaphoreType.DMA,
            pltpu.VMEM_SHARED((N_PAD, DH), jnp.float32),
        ],
    )
    return f(x0, x1, ep0, ep1, src, dst, zeros)


def _node_update_body(a0, a1, x0, x1, w_ref, b_ref, h0, h1):
    u0 = a0[...] + x0[...]
    u1 = a1[...] + x1[...]
    w = w_ref[...]
    acc = jnp.dot(u0, w[:DH, :], preferred_element_type=jnp.float32)
    acc = acc + jnp.dot(u1, w[DH:, :], preferred_element_type=jnp.float32)
    h = jnp.maximum(acc + b_ref[...], 0.0)
    h0[...] = h[:, :DH]
    h1[...] = h[:, DH:]


def _node_update(agg0, agg1, x0, x1, W, b):
    BN = 2000
    grid = (N // BN,)
    out = jax.ShapeDtypeStruct((N, DH), jnp.float32)
    half = pl.BlockSpec((BN, DH), lambda i: (i, 0))
    return pl.pallas_call(
        _node_update_body,
        grid=grid,
        in_specs=[half, half, half, half,
                  pl.BlockSpec((D, D), lambda i: (0, 0)),
                  pl.BlockSpec((1, D), lambda i: (0, 0))],
        out_specs=[half, half],
        out_shape=[out, out],
    )(agg0, agg1, x0, x1, W, b)


def _nu2_pool_body(batch_ref, a0, a1, x0, x1, w_ref, b_ref,
                   wl0, bl0, wl1, bl1, wemb, bemb, wout, bout, out_ref, acc):
    i = pl.program_id(0)

    @pl.when(i == 0)
    def _():
        acc[...] = jnp.zeros_like(acc)

    u0 = a0[...] + x0[...]
    u1 = a1[...] + x1[...]
    w = w_ref[...]
    hacc = jnp.dot(u0, w[:DH, :], preferred_element_type=jnp.float32)
    hacc = hacc + jnp.dot(u1, w[DH:, :], preferred_element_type=jnp.float32)
    h = jnp.maximum(hacc + b_ref[...], 0.0)

    bi = batch_ref[0, 0, :]
    gid = lax.broadcasted_iota(jnp.int32, (G, bi.shape[0]), 0)
    mask = (gid == bi[None, :]).astype(jnp.float32)
    acc[...] += jnp.dot(mask, h, preferred_element_type=jnp.float32)

    @pl.when(i == pl.num_programs(0) - 1)
    def _():
        p = acc[...]
        a = jnp.maximum(jnp.dot(p, wl0[...], preferred_element_type=jnp.float32) + bl0[...], 0.0)
        a = jnp.maximum(jnp.dot(a, wl1[...], preferred_element_type=jnp.float32) + bl1[...], 0.0)
        e = jnp.dot(a, wemb[...], preferred_element_type=jnp.float32) + bemb[...]
        out_ref[...] = jnp.dot(e, wout[...], preferred_element_type=jnp.float32) + bout[...]


def _nu2_pool(batch3d, agg0, agg1, h0, h1, W, b,
              Wl0, bl0, Wl1, bl1, Wemb, bemb, Wout_p, bout_p):
    BN = 2000
    grid = (N // BN,)
    half = pl.BlockSpec((BN, DH), lambda i: (i, 0))
    full = lambda r, c: pl.BlockSpec((r, c), lambda i: (0, 0))
    return pl.pallas_call(
        _nu2_pool_body,
        grid=grid,
        in_specs=[pl.BlockSpec((1, 1, BN), lambda i: (i, 0, 0)),
                  half, half, half, half,
                  full(D, D), full(1, D),
                  full(D, H), full(1, H),
                  full(H, H), full(1, H),
                  full(H, H), full(1, H),
                  full(H, 128), full(1, 128)],
        out_specs=pl.BlockSpec((G, 128), lambda i: (0, 0)),
        out_shape=jax.ShapeDtypeStruct((G, 128), jnp.float32),
        scratch_shapes=[pltpu.VMEM((G, D), jnp.float32)],
    )(batch3d, agg0, agg1, h0, h1, W, b,
      Wl0, bl0, Wl1, bl1, Wemb, bemb, Wout_p, bout_p)


def kernel(x, edge_index, edge_attr, batch, We0, W0, b0, We1, W1, b1,
           Wl0, bl0, Wl1, bl1, Wemb, bemb, Wout, bout):
    pad = E_PAD - E
    src = jnp.concatenate([edge_index[0], jnp.zeros((pad,), jnp.int32)])
    dst = jnp.concatenate([edge_index[1], jnp.full((pad,), N, jnp.int32)])
    ea = jnp.pad(edge_attr, ((0, pad), (0, 0)))
    x0 = x[:, :DH]
    x1 = x[:, DH:]
    zeros = jnp.zeros((ROWS_PER_TILE, DH), jnp.float32)

    ep00, ep01 = _edge_proj(ea, We0)

    a0, a1 = _sc_layer(x0, x1, ep00, ep01, src, dst, zeros)
    ep10, ep11 = _edge_proj(ea, We1)
    h0, h1 = _node_update(a0, a1, x0, x1, W0, b0.reshape(1, D))

    a0, a1 = _sc_layer(h0, h1, ep10, ep11, src, dst, zeros)
    Wout_p = jnp.pad(Wout, ((0, 0), (0, 128 - T)))
    bout_p = jnp.pad(bout, (0, 128 - T)).reshape(1, 128)
    out = _nu2_pool(batch.reshape(N // 2000, 1, 2000), a0, a1, h0, h1,
                    W1, b1.reshape(1, D),
                    Wl0, bl0.reshape(1, H), Wl1, bl1.reshape(1, H),
                    Wemb, bemb.reshape(1, H), Wout_p, bout_p)
    return out[:, :T]
